# baseline (speedup 1.0000x reference)
import functools

import jax
import jax.numpy as jnp
from jax import lax
from jax.experimental import pallas as pl
from jax.experimental.pallas import tpu as pltpu
from jax.experimental.pallas import tpu_sc as plsc

_NUM_CORES = 2
_NUM_SUBCORES = 16
_NW = _NUM_CORES * _NUM_SUBCORES
_IDX_MINOR = 128


@functools.cache
def _build(B, S, EMB):
  total = B * S
  chunk = total // _NW
  nsub = chunk // _IDX_MINOR
  assert chunk % _IDX_MINOR == 0 and S % chunk == 0

  mesh = plsc.VectorSubcoreMesh(core_axis_name="c", subcore_axis_name="s")

  @functools.partial(
      pl.kernel,
      out_type=jax.ShapeDtypeStruct((total, EMB), jnp.float32),
      mesh=mesh,
      scratch_types=[
          pltpu.VMEM((nsub, _IDX_MINOR), jnp.int32),
          pltpu.VMEM((chunk, EMB), jnp.float32),
          pltpu.SemaphoreType.DMA,
          pltpu.SemaphoreType.DMA,
      ],
  )
  def emb_kernel(src_hbm, word_hbm, pos_hbm, out_hbm, idx_v, rows_v, sem,
                 out_sem):
    wid = lax.axis_index("s") * _NUM_CORES + lax.axis_index("c")
    base = wid * chunk
    nseg = S // chunk
    b = wid // nseg
    s0 = lax.rem(wid, nseg) * chunk

    for j in range(nsub):
      pltpu.sync_copy(src_hbm.at[b, pl.ds(s0 + j * _IDX_MINOR, _IDX_MINOR)],
                      idx_v.at[j])
    pltpu.sync_copy(pos_hbm.at[pl.ds(s0, chunk)], rows_v)
    copies = [
        pltpu.async_copy(
            word_hbm.at[idx_v.at[j]],
            rows_v.at[pl.ds(j * _IDX_MINOR, _IDX_MINOR)],
            sem,
            add=True,
        )
        for j in range(nsub)
    ]
    outs = []
    for j in range(nsub):
      copies[j].wait()
      outs.append(
          pltpu.async_copy(
              rows_v.at[pl.ds(j * _IDX_MINOR, _IDX_MINOR)],
              out_hbm.at[pl.ds(base + j * _IDX_MINOR, _IDX_MINOR)],
              out_sem,
          ))
    for oc in outs:
      oc.wait()

  return emb_kernel


def kernel(src, word_table, pos_table):
  B, S = src.shape
  EMB = word_table.shape[1]
  fn = _build(B, S, EMB)
  out = fn(src.astype(jnp.int32), word_table, pos_table)
  return out.reshape(B, S, EMB)


# speedup vs baseline: 1.0354x; 1.0354x over previous
import functools

import jax
import jax.numpy as jnp
from jax import lax
from jax.experimental import pallas as pl
from jax.experimental.pallas import tpu as pltpu
from jax.experimental.pallas import tpu_sc as plsc

_NUM_CORES = 2
_NUM_SUBCORES = 16
_NW = _NUM_CORES * _NUM_SUBCORES
_IDX_MINOR = 128


@functools.cache
def _build(B, S, EMB):
  total = B * S
  chunk = total // _NW
  nsub = chunk // _IDX_MINOR
  assert chunk % _IDX_MINOR == 0 and S % chunk == 0

  mesh = plsc.VectorSubcoreMesh(core_axis_name="c", subcore_axis_name="s")

  @functools.partial(
      pl.kernel,
      out_type=jax.ShapeDtypeStruct((total, EMB), jnp.float32),
      mesh=mesh,
      scratch_types=[
          pltpu.VMEM((chunk,), jnp.int32),
          pltpu.VMEM((chunk, EMB), jnp.float32),
          pltpu.SemaphoreType.DMA,
          pltpu.SemaphoreType.DMA,
          pltpu.SemaphoreType.DMA,
      ],
  )
  def emb_kernel(src_hbm, word_hbm, pos_hbm, out_hbm, idx_v, rows_v, sem,
                 out_sem, idx_sem):
    wid = lax.axis_index("s") * _NUM_CORES + lax.axis_index("c")
    base = wid * chunk
    nseg = S // chunk
    b = wid // nseg
    s0 = lax.rem(wid, nseg) * chunk

    idx_cp = pltpu.async_copy(src_hbm.at[b, pl.ds(s0, chunk)], idx_v, idx_sem)
    pltpu.sync_copy(pos_hbm.at[pl.ds(s0, chunk)], rows_v)
    idx_cp.wait()
    copies = [
        pltpu.async_copy(
            word_hbm.at[idx_v.at[pl.ds(j * _IDX_MINOR, _IDX_MINOR)]],
            rows_v.at[pl.ds(j * _IDX_MINOR, _IDX_MINOR)],
            sem,
            add=True,
        )
        for j in range(nsub)
    ]
    outs = []
    for j in range(nsub):
      copies[j].wait()
      outs.append(
          pltpu.async_copy(
              rows_v.at[pl.ds(j * _IDX_MINOR, _IDX_MINOR)],
              out_hbm.at[pl.ds(base + j * _IDX_MINOR, _IDX_MINOR)],
              out_sem,
          ))
    for oc in outs:
      oc.wait()

  return emb_kernel


def kernel(src, word_table, pos_table):
  B, S = src.shape
  EMB = word_table.shape[1]
  fn = _build(B, S, EMB)
  out = fn(src.astype(jnp.int32), word_table, pos_table)
  return out.reshape(B, S, EMB)
